# baseline (speedup 1.0000x reference)
import functools

import jax
import jax.numpy as jnp
from jax import lax
from jax.experimental import pallas as pl
from jax.experimental.pallas import tpu as pltpu
from jax.experimental.pallas import tpu_sc as plsc

NN = 10000
DD = 128
EE = 320000
NC = 2
NS = 16
NW = NC * NS
CHUNK = 64
CH = 160
NCHUNK = NW * CH
EPAD = NCHUNK * CHUNK
CH0 = 240
CH1 = 80
C0TOT = NS * CH0
NCHUNK_A = NS * (CH0 + CH1)
NROW_A = NCHUNK_A + (CH0 - CH1)
EPAD_A = NROW_A * CHUNK
NPAD = 10112
RPT = NPAD // NS
ZR = 8
NBUF = 2
RB = NPAD // 8


def _sc_mesh():
    return plsc.VectorSubcoreMesh(
        core_axis_name="c", subcore_axis_name="s", num_cores=NC, num_subcores=NS
    )


def _sc_count(idx2d):
    @functools.partial(
        pl.kernel,
        out_type=jax.ShapeDtypeStruct((NC, NPAD, 16), jnp.float32),
        mesh=_sc_mesh(),
        compiler_params=pltpu.CompilerParams(use_tc_tiling_on_sc=False),
        scratch_types=[
            pltpu.VMEM((CH, CHUNK), jnp.int32),
            pltpu.VMEM((CHUNK, 16), jnp.float32),
            pltpu.VMEM((ZR, 16), jnp.float32),
            pltpu.VMEM_SHARED((NPAD, 16), jnp.float32),
        ],
    )
    def cnt_kernel(idx_hbm, cnt_hbm, idx_v, ones_v, zero_v, acc):
        c = lax.axis_index("c")
        s = lax.axis_index("s")
        wid = s * NC + c
        ones16 = jnp.ones((16,), jnp.float32)
        zeros16 = jnp.zeros((16,), jnp.float32)

        def fill_ones(i, carry):
            ones_v[i] = ones16
            return carry

        lax.fori_loop(0, CHUNK, fill_ones, 0)

        def fill_zeros(i, carry):
            zero_v[i] = zeros16
            return carry

        lax.fori_loop(0, ZR, fill_zeros, 0)

        base = s * RPT

        def zinit(i, carry):
            pltpu.sync_copy(zero_v, acc.at[pl.ds(base + i * ZR, ZR)])
            return carry

        lax.fori_loop(0, RPT // ZR, zinit, 0)
        plsc.subcore_barrier()

        pltpu.sync_copy(idx_hbm.at[pl.ds(wid * CH, CH)], idx_v)

        def body(j, carry):
            pltpu.sync_copy(ones_v, acc.at[idx_v.at[j]], add=True)
            return carry

        lax.fori_loop(0, CH, body, 0)
        plsc.subcore_barrier()

        pltpu.sync_copy(acc.at[pl.ds(base, RPT)],
                        cnt_hbm.at[c, pl.ds(base, RPT)])

    return cnt_kernel(idx2d)


def _sc_agg(y, src2d, dst2d):
    @functools.partial(
        pl.kernel,
        out_type=jax.ShapeDtypeStruct((NC, NPAD, DD), jnp.float32),
        mesh=_sc_mesh(),
        compiler_params=pltpu.CompilerParams(use_tc_tiling_on_sc=False),
        scratch_types=[
            pltpu.VMEM((CH0, CHUNK), jnp.int32),
            pltpu.VMEM((CH0, CHUNK), jnp.int32),
            pltpu.VMEM_SHARED((NPAD, DD), jnp.float32),
            [pltpu.SemaphoreType.DMA] * 2,
        ],
    )
    def agg_kernel(y_hbm, src_hbm, dst_hbm, p_hbm, src_v, dst_v, acc, gsems):
        c = lax.axis_index("c")
        s = lax.axis_index("s")
        is0 = c == 0
        mych = jnp.where(is0, CH0, CH1)
        mybase = jnp.where(is0, s * CH0, C0TOT + s * CH1)
        zeros16 = jnp.zeros((16,), jnp.float32)

        def scoped(rows0, rows1):
            rows = (rows0, rows1)

            def fz(i, carry):
                rows0[i // 8, pl.ds((i % 8) * 16, 16)] = zeros16
                return carry

            lax.fori_loop(0, ZR * 8, fz, 0)

            base = s * RPT
            zsrc = rows0.at[pl.ds(0, ZR)]

            def zinit(i, carry):
                pltpu.sync_copy(zsrc, acc.at[pl.ds(base + i * ZR, ZR)])
                return carry

            lax.fori_loop(0, RPT // ZR, zinit, 0)
            plsc.subcore_barrier()

            pltpu.sync_copy(src_hbm.at[pl.ds(mybase, CH0)], src_v)
            pltpu.sync_copy(dst_hbm.at[pl.ds(mybase, CH0)], dst_v)

            pltpu.async_copy(y_hbm.at[src_v.at[0]], rows0, gsems[0])
            pltpu.async_copy(y_hbm.at[src_v.at[1]], rows1, gsems[1])

            def body(g, carry):
                for b in range(2):
                    j = g * 2 + b
                    pltpu.make_async_copy(y_hbm.at[src_v.at[j]], rows[b],
                                          gsems[b]).wait()
                    pltpu.sync_copy(rows[b], acc.at[dst_v.at[j]], add=True)

                    @pl.when(j + 2 < mych)
                    def _():
                        pltpu.async_copy(y_hbm.at[src_v.at[j + 2]], rows[b],
                                         gsems[b])

                return carry

            lax.fori_loop(0, mych // 2, body, 0)
            plsc.subcore_barrier()

            pltpu.sync_copy(acc.at[pl.ds(base, RPT)],
                            p_hbm.at[c, pl.ds(base, RPT)])

        pl.run_scoped(
            scoped,
            pltpu.VMEM((CHUNK, DD), jnp.float32),
            pltpu.VMEM((CHUNK, DD), jnp.float32),
        )

    return agg_kernel(y, src2d, dst2d)


def _tc_norm_matmul(xp, W1, cs0, cs1, cd0, cd1):
    def body(x_ref, w_ref, cs0_ref, cs1_ref, cd0_ref, cd1_ref,
             y_ref, so_ref, si_ref):
        deg_o = jnp.maximum(cs0_ref[:, 0:1] + cs1_ref[:, 0:1], 1.0)
        deg_i = jnp.maximum(cd0_ref[:, 0:1] + cd1_ref[:, 0:1], 1.0)
        so = jnp.broadcast_to(lax.rsqrt(deg_o), (RB, DD))
        si = jnp.broadcast_to(lax.rsqrt(deg_i), (RB, DD))
        y = jnp.dot(x_ref[...], w_ref[...], preferred_element_type=jnp.float32,
                    precision=lax.Precision.HIGHEST)
        y_ref[...] = y * so
        so_ref[...] = so
        si_ref[...] = si

    row = pl.BlockSpec((RB, DD), lambda i: (i, 0))
    cnt = pl.BlockSpec((RB, 16), lambda i: (i, 0))
    full = pl.BlockSpec((DD, DD), lambda i: (0, 0))
    shape = jax.ShapeDtypeStruct((NPAD, DD), jnp.float32)
    return pl.pallas_call(
        body,
        grid=(NPAD // RB,),
        in_specs=[row, full, cnt, cnt, cnt, cnt],
        out_specs=[row, row, row],
        out_shape=[shape, shape, shape],
    )(xp, W1, cs0, cs1, cd0, cd1)


def _tc_mid(p0, p1, si, so, b1, W2):
    def body(p0_ref, p1_ref, si_ref, so_ref, b_ref, w_ref, y_ref):
        agg = (p0_ref[...] + p1_ref[...]) * si_ref[...]
        h = agg + b_ref[...]
        h = jnp.where(h >= 0, h, h * jnp.float32(0.01))
        y = jnp.dot(h, w_ref[...], preferred_element_type=jnp.float32,
                    precision=lax.Precision.HIGHEST)
        y_ref[...] = y * so_ref[...]

    row = pl.BlockSpec((RB, DD), lambda i: (i, 0))
    bias = pl.BlockSpec((1, DD), lambda i: (0, 0))
    full = pl.BlockSpec((DD, DD), lambda i: (0, 0))
    return pl.pallas_call(
        body,
        grid=(NPAD // RB,),
        in_specs=[row, row, row, row, bias, full],
        out_specs=row,
        out_shape=jax.ShapeDtypeStruct((NPAD, DD), jnp.float32),
    )(p0, p1, si, so, b1, W2)


def _tc_final(p0, p1, si, b2):
    def body(p0_ref, p1_ref, si_ref, b_ref, y_ref):
        y_ref[...] = (p0_ref[...] + p1_ref[...]) * si_ref[...] + b_ref[...]

    row = pl.BlockSpec((RB, DD), lambda i: (i, 0))
    bias = pl.BlockSpec((1, DD), lambda i: (0, 0))
    return pl.pallas_call(
        body,
        grid=(NPAD // RB,),
        in_specs=[row, row, row, bias],
        out_specs=row,
        out_shape=jax.ShapeDtypeStruct((NPAD, DD), jnp.float32),
    )(p0, p1, si, b2)


def kernel(x, edge_index, W1, b1, W2, b2):
    src = edge_index[0]
    dst = edge_index[1]
    fill = jnp.full((EPAD - EE,), NN, dtype=jnp.int32)
    src2d = jnp.concatenate([src, fill]).reshape(NCHUNK, CHUNK)
    dst2d = jnp.concatenate([dst, fill]).reshape(NCHUNK, CHUNK)
    fill_a = jnp.full((EPAD_A - EE,), NN, dtype=jnp.int32)
    src2a = jnp.concatenate([src, fill_a]).reshape(NROW_A, CHUNK)
    dst2a = jnp.concatenate([dst, fill_a]).reshape(NROW_A, CHUNK)
    xp = jnp.zeros((NPAD, DD), jnp.float32).at[:NN].set(x)

    cnt_s = _sc_count(src2d)
    cnt_d = _sc_count(dst2d)
    y1, so, si = _tc_norm_matmul(xp, W1, cnt_s[0], cnt_s[1],
                                 cnt_d[0], cnt_d[1])
    p1 = _sc_agg(y1, src2a, dst2a)
    y2 = _tc_mid(p1[0], p1[1], si, so, b1.reshape(1, DD), W2)
    p2 = _sc_agg(y2, src2a, dst2a)
    out = _tc_final(p2[0], p2[1], si, b2.reshape(1, DD))
    return out[:NN]


# speedup vs baseline: 1.0098x; 1.0098x over previous
import functools

import jax
import jax.numpy as jnp
from jax import lax
from jax.experimental import pallas as pl
from jax.experimental.pallas import tpu as pltpu
from jax.experimental.pallas import tpu_sc as plsc

NN = 10000
DD = 128
EE = 320000
NC = 2
NS = 16
NW = NC * NS
CHUNK = 64
CH = 160
NCHUNK = NW * CH
EPAD = NCHUNK * CHUNK
CH0 = 240
CH1 = 80
C0TOT = NS * CH0
NCHUNK_A = NS * (CH0 + CH1)
NROW_A = NCHUNK_A + (CH0 - CH1)
EPAD_A = NROW_A * CHUNK
NPAD = 10112
RPT = NPAD // NS
ZR = 8
NBUF = 2
RB = NPAD // 8


def _sc_mesh():
    return plsc.VectorSubcoreMesh(
        core_axis_name="c", subcore_axis_name="s", num_cores=NC, num_subcores=NS
    )


def _sc_count(idx2d):
    @functools.partial(
        pl.kernel,
        out_type=jax.ShapeDtypeStruct((NC, NPAD, 16), jnp.float32),
        mesh=_sc_mesh(),
        compiler_params=pltpu.CompilerParams(use_tc_tiling_on_sc=False),
        scratch_types=[
            pltpu.VMEM((CH, CHUNK), jnp.int32),
            pltpu.VMEM((CHUNK, 16), jnp.float32),
            pltpu.VMEM((ZR, 16), jnp.float32),
            pltpu.VMEM_SHARED((NPAD, 16), jnp.float32),
        ],
    )
    def cnt_kernel(idx_hbm, cnt_hbm, idx_v, ones_v, zero_v, acc):
        c = lax.axis_index("c")
        s = lax.axis_index("s")
        wid = s * NC + c
        ones16 = jnp.ones((16,), jnp.float32)
        zeros16 = jnp.zeros((16,), jnp.float32)

        def fill_ones(i, carry):
            ones_v[i] = ones16
            return carry

        lax.fori_loop(0, CHUNK, fill_ones, 0)

        def fill_zeros(i, carry):
            zero_v[i] = zeros16
            return carry

        lax.fori_loop(0, ZR, fill_zeros, 0)

        base = s * RPT

        def zinit(i, carry):
            pltpu.sync_copy(zero_v, acc.at[pl.ds(base + i * ZR, ZR)])
            return carry

        lax.fori_loop(0, RPT // ZR, zinit, 0)
        plsc.subcore_barrier()

        pltpu.sync_copy(idx_hbm.at[pl.ds(wid * CH, CH)], idx_v)

        def body(j, carry):
            pltpu.sync_copy(ones_v, acc.at[idx_v.at[j]], add=True)
            return carry

        lax.fori_loop(0, CH, body, 0)
        plsc.subcore_barrier()

        pltpu.sync_copy(acc.at[pl.ds(base, RPT)],
                        cnt_hbm.at[c, pl.ds(base, RPT)])

    return cnt_kernel(idx2d)


def _sc_agg(y, src2d, dst2d):
    @functools.partial(
        pl.kernel,
        out_type=jax.ShapeDtypeStruct((NC, NPAD, DD), jnp.float32),
        mesh=_sc_mesh(),
        compiler_params=pltpu.CompilerParams(use_tc_tiling_on_sc=False),
        scratch_types=[
            pltpu.VMEM((CH0, CHUNK), jnp.int32),
            pltpu.VMEM((CH0, CHUNK), jnp.int32),
            pltpu.VMEM_SHARED((NPAD, DD), jnp.float32),
            [pltpu.SemaphoreType.DMA] * 2,
        ],
    )
    def agg_kernel(y_hbm, src_hbm, dst_hbm, p_hbm, src_v, dst_v, acc, gsems):
        c = lax.axis_index("c")
        s = lax.axis_index("s")
        is0 = c == 0
        mych = jnp.where(is0, CH0, CH1)
        mybase = jnp.where(is0, s * CH0, C0TOT + s * CH1)
        zeros16 = jnp.zeros((16,), jnp.float32)

        def scoped(rows0, rows1):
            rows = (rows0, rows1)

            def fz(i, carry):
                rows0[i // 8, pl.ds((i % 8) * 16, 16)] = zeros16
                return carry

            lax.fori_loop(0, ZR * 8, fz, 0)

            base = s * RPT
            zsrc = rows0.at[pl.ds(0, ZR)]

            def zinit(i, carry):
                pltpu.async_copy(zsrc, acc.at[pl.ds(base + i * ZR, ZR)],
                                 gsems[0])
                return carry

            lax.fori_loop(0, RPT // ZR, zinit, 0)
            pltpu.async_copy(src_hbm.at[pl.ds(mybase, CH0)], src_v, gsems[1])

            def zdrain(i, carry):
                pltpu.make_async_copy(zsrc, acc.at[pl.ds(base + i * ZR, ZR)],
                                      gsems[0]).wait()
                return carry

            lax.fori_loop(0, RPT // ZR, zdrain, 0)
            plsc.subcore_barrier()

            pltpu.make_async_copy(src_hbm.at[pl.ds(mybase, CH0)], src_v,
                                  gsems[1]).wait()
            pltpu.sync_copy(dst_hbm.at[pl.ds(mybase, CH0)], dst_v)

            pltpu.async_copy(y_hbm.at[src_v.at[0]], rows0, gsems[0])
            pltpu.async_copy(y_hbm.at[src_v.at[1]], rows1, gsems[1])

            def body(g, carry):
                for b in range(2):
                    j = g * 2 + b
                    pltpu.make_async_copy(y_hbm.at[src_v.at[j]], rows[b],
                                          gsems[b]).wait()
                    pltpu.sync_copy(rows[b], acc.at[dst_v.at[j]], add=True)

                    @pl.when(j + 2 < mych)
                    def _():
                        pltpu.async_copy(y_hbm.at[src_v.at[j + 2]], rows[b],
                                         gsems[b])

                return carry

            lax.fori_loop(0, mych // 2, body, 0)
            plsc.subcore_barrier()

            pltpu.sync_copy(acc.at[pl.ds(base, RPT)],
                            p_hbm.at[c, pl.ds(base, RPT)])

        pl.run_scoped(
            scoped,
            pltpu.VMEM((CHUNK, DD), jnp.float32),
            pltpu.VMEM((CHUNK, DD), jnp.float32),
        )

    return agg_kernel(y, src2d, dst2d)


def _tc_norm_matmul(xp, W1, cs0, cs1, cd0, cd1):
    def body(x_ref, w_ref, cs0_ref, cs1_ref, cd0_ref, cd1_ref,
             y_ref, so_ref, si_ref):
        deg_o = jnp.maximum(cs0_ref[:, 0:1] + cs1_ref[:, 0:1], 1.0)
        deg_i = jnp.maximum(cd0_ref[:, 0:1] + cd1_ref[:, 0:1], 1.0)
        so = jnp.broadcast_to(lax.rsqrt(deg_o), (RB, DD))
        si = jnp.broadcast_to(lax.rsqrt(deg_i), (RB, DD))
        y = jnp.dot(x_ref[...], w_ref[...], preferred_element_type=jnp.float32,
                    precision=lax.Precision.HIGHEST)
        y_ref[...] = y * so
        so_ref[...] = so
        si_ref[...] = si

    row = pl.BlockSpec((RB, DD), lambda i: (i, 0))
    cnt = pl.BlockSpec((RB, 16), lambda i: (i, 0))
    full = pl.BlockSpec((DD, DD), lambda i: (0, 0))
    shape = jax.ShapeDtypeStruct((NPAD, DD), jnp.float32)
    return pl.pallas_call(
        body,
        grid=(NPAD // RB,),
        in_specs=[row, full, cnt, cnt, cnt, cnt],
        out_specs=[row, row, row],
        out_shape=[shape, shape, shape],
    )(xp, W1, cs0, cs1, cd0, cd1)


def _tc_mid(p0, p1, si, so, b1, W2):
    def body(p0_ref, p1_ref, si_ref, so_ref, b_ref, w_ref, y_ref):
        agg = (p0_ref[...] + p1_ref[...]) * si_ref[...]
        h = agg + b_ref[...]
        h = jnp.where(h >= 0, h, h * jnp.float32(0.01))
        y = jnp.dot(h, w_ref[...], preferred_element_type=jnp.float32,
                    precision=lax.Precision.HIGHEST)
        y_ref[...] = y * so_ref[...]

    row = pl.BlockSpec((RB, DD), lambda i: (i, 0))
    bias = pl.BlockSpec((1, DD), lambda i: (0, 0))
    full = pl.BlockSpec((DD, DD), lambda i: (0, 0))
    return pl.pallas_call(
        body,
        grid=(NPAD // RB,),
        in_specs=[row, row, row, row, bias, full],
        out_specs=row,
        out_shape=jax.ShapeDtypeStruct((NPAD, DD), jnp.float32),
    )(p0, p1, si, so, b1, W2)


def _tc_final(p0, p1, si, b2):
    def body(p0_ref, p1_ref, si_ref, b_ref, y_ref):
        y_ref[...] = (p0_ref[...] + p1_ref[...]) * si_ref[...] + b_ref[...]

    row = pl.BlockSpec((RB, DD), lambda i: (i, 0))
    bias = pl.BlockSpec((1, DD), lambda i: (0, 0))
    return pl.pallas_call(
        body,
        grid=(NPAD // RB,),
        in_specs=[row, row, row, bias],
        out_specs=row,
        out_shape=jax.ShapeDtypeStruct((NPAD, DD), jnp.float32),
    )(p0, p1, si, b2)


def kernel(x, edge_index, W1, b1, W2, b2):
    src = edge_index[0]
    dst = edge_index[1]
    fill = jnp.full((EPAD - EE,), NN, dtype=jnp.int32)
    src2d = jnp.concatenate([src, fill]).reshape(NCHUNK, CHUNK)
    dst2d = jnp.concatenate([dst, fill]).reshape(NCHUNK, CHUNK)
    fill_a = jnp.full((EPAD_A - EE,), NN, dtype=jnp.int32)
    src2a = jnp.concatenate([src, fill_a]).reshape(NROW_A, CHUNK)
    dst2a = jnp.concatenate([dst, fill_a]).reshape(NROW_A, CHUNK)
    xp = jnp.zeros((NPAD, DD), jnp.float32).at[:NN].set(x)

    cnt_s = _sc_count(src2d)
    cnt_d = _sc_count(dst2d)
    y1, so, si = _tc_norm_matmul(xp, W1, cnt_s[0], cnt_s[1],
                                 cnt_d[0], cnt_d[1])
    p1 = _sc_agg(y1, src2a, dst2a)
    y2 = _tc_mid(p1[0], p1[1], si, so, b1.reshape(1, DD), W2)
    p2 = _sc_agg(y2, src2a, dst2a)
    out = _tc_final(p2[0], p2[1], si, b2.reshape(1, DD))
    return out[:NN]


# speedup vs baseline: 1.0504x; 1.0402x over previous
import functools

import jax
import jax.numpy as jnp
from jax import lax
from jax.experimental import pallas as pl
from jax.experimental.pallas import tpu as pltpu
from jax.experimental.pallas import tpu_sc as plsc

NN = 10000
DD = 128
EE = 320000
NC = 2
NS = 16
NW = NC * NS
CHUNK = 80
CH = 128
NCHUNK = NW * CH
EPAD = NCHUNK * CHUNK
CH0 = 128
CH1 = 128
C0TOT = NS * CH0
NCHUNK_A = NS * (CH0 + CH1)
NROW_A = NCHUNK_A + (CH0 - CH1)
EPAD_A = NROW_A * CHUNK
NPAD = 10112
RPT = NPAD // NS
ZR = 8
NBUF = 2
RB = NPAD // 8


def _sc_mesh():
    return plsc.VectorSubcoreMesh(
        core_axis_name="c", subcore_axis_name="s", num_cores=NC, num_subcores=NS
    )


def _sc_count(idx2d):
    @functools.partial(
        pl.kernel,
        out_type=jax.ShapeDtypeStruct((NC, NPAD, 16), jnp.float32),
        mesh=_sc_mesh(),
        compiler_params=pltpu.CompilerParams(use_tc_tiling_on_sc=False),
        scratch_types=[
            pltpu.VMEM((CH, CHUNK), jnp.int32),
            pltpu.VMEM((CHUNK, 16), jnp.float32),
            pltpu.VMEM((ZR, 16), jnp.float32),
            pltpu.VMEM_SHARED((NPAD, 16), jnp.float32),
        ],
    )
    def cnt_kernel(idx_hbm, cnt_hbm, idx_v, ones_v, zero_v, acc):
        c = lax.axis_index("c")
        s = lax.axis_index("s")
        wid = s * NC + c
        ones16 = jnp.ones((16,), jnp.float32)
        zeros16 = jnp.zeros((16,), jnp.float32)

        def fill_ones(i, carry):
            ones_v[i] = ones16
            return carry

        lax.fori_loop(0, CHUNK, fill_ones, 0)

        def fill_zeros(i, carry):
            zero_v[i] = zeros16
            return carry

        lax.fori_loop(0, ZR, fill_zeros, 0)

        base = s * RPT

        def zinit(i, carry):
            pltpu.sync_copy(zero_v, acc.at[pl.ds(base + i * ZR, ZR)])
            return carry

        lax.fori_loop(0, RPT // ZR, zinit, 0)
        plsc.subcore_barrier()

        pltpu.sync_copy(idx_hbm.at[pl.ds(wid * CH, CH)], idx_v)

        def body(j, carry):
            pltpu.sync_copy(ones_v, acc.at[idx_v.at[j]], add=True)
            return carry

        lax.fori_loop(0, CH, body, 0)
        plsc.subcore_barrier()

        pltpu.sync_copy(acc.at[pl.ds(base, RPT)],
                        cnt_hbm.at[c, pl.ds(base, RPT)])

    return cnt_kernel(idx2d)


def _sc_agg(y, src2d, dst2d):
    @functools.partial(
        pl.kernel,
        out_type=jax.ShapeDtypeStruct((NC, NPAD, DD), jnp.float32),
        mesh=_sc_mesh(),
        compiler_params=pltpu.CompilerParams(use_tc_tiling_on_sc=False),
        scratch_types=[
            pltpu.VMEM((CH0, CHUNK), jnp.int32),
            pltpu.VMEM((CH0, CHUNK), jnp.int32),
            pltpu.VMEM_SHARED((NPAD, DD), jnp.float32),
            [pltpu.SemaphoreType.DMA] * 2,
        ],
    )
    def agg_kernel(y_hbm, src_hbm, dst_hbm, p_hbm, src_v, dst_v, acc, gsems):
        c = lax.axis_index("c")
        s = lax.axis_index("s")
        is0 = c == 0
        mych = jnp.where(is0, CH0, CH1)
        mybase = jnp.where(is0, s * CH0, C0TOT + s * CH1)
        zeros16 = jnp.zeros((16,), jnp.float32)

        def scoped(rows0, rows1):
            rows = (rows0, rows1)

            def fz(i, carry):
                rows0[i // 8, pl.ds((i % 8) * 16, 16)] = zeros16
                return carry

            lax.fori_loop(0, ZR * 8, fz, 0)

            base = s * RPT
            zsrc = rows0.at[pl.ds(0, ZR)]

            def zinit(i, carry):
                pltpu.async_copy(zsrc, acc.at[pl.ds(base + i * ZR, ZR)],
                                 gsems[0])
                return carry

            lax.fori_loop(0, RPT // ZR, zinit, 0)
            pltpu.async_copy(src_hbm.at[pl.ds(mybase, CH0)], src_v, gsems[1])

            def zdrain(i, carry):
                pltpu.make_async_copy(zsrc, acc.at[pl.ds(base + i * ZR, ZR)],
                                      gsems[0]).wait()
                return carry

            lax.fori_loop(0, RPT // ZR, zdrain, 0)
            plsc.subcore_barrier()

            pltpu.make_async_copy(src_hbm.at[pl.ds(mybase, CH0)], src_v,
                                  gsems[1]).wait()
            pltpu.sync_copy(dst_hbm.at[pl.ds(mybase, CH0)], dst_v)

            pltpu.async_copy(y_hbm.at[src_v.at[0]], rows0, gsems[0])
            pltpu.async_copy(y_hbm.at[src_v.at[1]], rows1, gsems[1])

            def body(g, carry):
                for b in range(2):
                    j = g * 2 + b
                    pltpu.make_async_copy(y_hbm.at[src_v.at[j]], rows[b],
                                          gsems[b]).wait()
                    pltpu.sync_copy(rows[b], acc.at[dst_v.at[j]], add=True)

                    @pl.when(j + 2 < mych)
                    def _():
                        pltpu.async_copy(y_hbm.at[src_v.at[j + 2]], rows[b],
                                         gsems[b])

                return carry

            lax.fori_loop(0, mych // 2, body, 0)
            plsc.subcore_barrier()

            pltpu.sync_copy(acc.at[pl.ds(base, RPT)],
                            p_hbm.at[c, pl.ds(base, RPT)])

        pl.run_scoped(
            scoped,
            pltpu.VMEM((CHUNK, DD), jnp.float32),
            pltpu.VMEM((CHUNK, DD), jnp.float32),
        )

    return agg_kernel(y, src2d, dst2d)


def _tc_norm_matmul(xp, W1, cs0, cs1, cd0, cd1):
    def body(x_ref, w_ref, cs0_ref, cs1_ref, cd0_ref, cd1_ref,
             y_ref, so_ref, si_ref):
        deg_o = jnp.maximum(cs0_ref[:, 0:1] + cs1_ref[:, 0:1], 1.0)
        deg_i = jnp.maximum(cd0_ref[:, 0:1] + cd1_ref[:, 0:1], 1.0)
        so = jnp.broadcast_to(lax.rsqrt(deg_o), (RB, DD))
        si = jnp.broadcast_to(lax.rsqrt(deg_i), (RB, DD))
        y = jnp.dot(x_ref[...], w_ref[...], preferred_element_type=jnp.float32,
                    precision=lax.Precision.HIGHEST)
        y_ref[...] = y * so
        so_ref[...] = so
        si_ref[...] = si

    row = pl.BlockSpec((RB, DD), lambda i: (i, 0))
    cnt = pl.BlockSpec((RB, 16), lambda i: (i, 0))
    full = pl.BlockSpec((DD, DD), lambda i: (0, 0))
    shape = jax.ShapeDtypeStruct((NPAD, DD), jnp.float32)
    return pl.pallas_call(
        body,
        grid=(NPAD // RB,),
        in_specs=[row, full, cnt, cnt, cnt, cnt],
        out_specs=[row, row, row],
        out_shape=[shape, shape, shape],
    )(xp, W1, cs0, cs1, cd0, cd1)


def _tc_mid(p0, p1, si, so, b1, W2):
    def body(p0_ref, p1_ref, si_ref, so_ref, b_ref, w_ref, y_ref):
        agg = (p0_ref[...] + p1_ref[...]) * si_ref[...]
        h = agg + b_ref[...]
        h = jnp.where(h >= 0, h, h * jnp.float32(0.01))
        y = jnp.dot(h, w_ref[...], preferred_element_type=jnp.float32,
                    precision=lax.Precision.HIGHEST)
        y_ref[...] = y * so_ref[...]

    row = pl.BlockSpec((RB, DD), lambda i: (i, 0))
    bias = pl.BlockSpec((1, DD), lambda i: (0, 0))
    full = pl.BlockSpec((DD, DD), lambda i: (0, 0))
    return pl.pallas_call(
        body,
        grid=(NPAD // RB,),
        in_specs=[row, row, row, row, bias, full],
        out_specs=row,
        out_shape=jax.ShapeDtypeStruct((NPAD, DD), jnp.float32),
    )(p0, p1, si, so, b1, W2)


def _tc_final(p0, p1, si, b2):
    def body(p0_ref, p1_ref, si_ref, b_ref, y_ref):
        y_ref[...] = (p0_ref[...] + p1_ref[...]) * si_ref[...] + b_ref[...]

    row = pl.BlockSpec((RB, DD), lambda i: (i, 0))
    bias = pl.BlockSpec((1, DD), lambda i: (0, 0))
    return pl.pallas_call(
        body,
        grid=(NPAD // RB,),
        in_specs=[row, row, row, bias],
        out_specs=row,
        out_shape=jax.ShapeDtypeStruct((NPAD, DD), jnp.float32),
    )(p0, p1, si, b2)


def kernel(x, edge_index, W1, b1, W2, b2):
    src = edge_index[0]
    dst = edge_index[1]
    fill = jnp.full((EPAD - EE,), NN, dtype=jnp.int32)
    src2d = jnp.concatenate([src, fill]).reshape(NCHUNK, CHUNK)
    dst2d = jnp.concatenate([dst, fill]).reshape(NCHUNK, CHUNK)
    fill_a = jnp.full((EPAD_A - EE,), NN, dtype=jnp.int32)
    src2a = jnp.concatenate([src, fill_a]).reshape(NROW_A, CHUNK)
    dst2a = jnp.concatenate([dst, fill_a]).reshape(NROW_A, CHUNK)
    xp = jnp.zeros((NPAD, DD), jnp.float32).at[:NN].set(x)

    cnt_s = _sc_count(src2d)
    cnt_d = _sc_count(dst2d)
    y1, so, si = _tc_norm_matmul(xp, W1, cnt_s[0], cnt_s[1],
                                 cnt_d[0], cnt_d[1])
    p1 = _sc_agg(y1, src2a, dst2a)
    y2 = _tc_mid(p1[0], p1[1], si, so, b1.reshape(1, DD), W2)
    p2 = _sc_agg(y2, src2a, dst2a)
    out = _tc_final(p2[0], p2[1], si, b2.reshape(1, DD))
    return out[:NN]
